# baseline (speedup 1.0000x reference)
import functools

import jax
import jax.numpy as jnp
from jax.experimental import pallas as pl
from jax.experimental.pallas import tpu as pltpu

D_MODEL = 1024
D_SPACE = 64
N_FQK = 2048
N_FV = 1024
N_REL = 2048
N_VAL = 1024
_POOLS = (
    (0, N_FQK, 64),
    (N_FQK, N_FV, 32),
    (N_FQK + N_FV, N_REL, 64),
    (N_FQK + N_FV + N_REL + N_VAL, N_REL, 64),
    (N_FQK + N_FV + N_REL, N_VAL, 32),
)
_N_TOTAL = N_FQK + N_FV + 2 * N_REL + N_VAL


def _logits_body(x_ref, w_ref, b_ref, emb_ref, out_ref):
    h = jnp.dot(x_ref[...], w_ref[...], preferred_element_type=jnp.float32)
    h = h + b_ref[...]
    e = emb_ref[...]
    inv = jax.lax.rsqrt(jnp.maximum(jnp.sum(e * e, axis=1, keepdims=True), 1e-24))
    out_ref[...] = jnp.dot(h, (e * inv).T, preferred_element_type=jnp.float32)


def kernel(x, W_proj, b_proj, neuron_emb, neuron_emb_rk):
    B, S, _ = x.shape
    T = B * S
    xf = x.reshape(T, D_MODEL)
    emb = jnp.concatenate(
        [neuron_emb[: N_FQK + N_FV + N_REL + N_VAL], neuron_emb_rk], axis=0
    )
    TB = 256
    logits = pl.pallas_call(
        _logits_body,
        grid=(T // TB,),
        in_specs=[
            pl.BlockSpec((TB, D_MODEL), lambda i: (i, 0)),
            pl.BlockSpec((D_MODEL, D_SPACE), lambda i: (0, 0)),
            pl.BlockSpec((1, D_SPACE), lambda i: (0, 0)),
            pl.BlockSpec((_N_TOTAL, D_SPACE), lambda i: (0, 0)),
        ],
        out_specs=pl.BlockSpec((TB, _N_TOTAL), lambda i: (i, 0)),
        out_shape=jax.ShapeDtypeStruct((T, _N_TOTAL), jnp.float32),
    )(xf, W_proj, b_proj.reshape(1, D_SPACE), emb)

    outs = []
    for (s, n, k) in _POOLS:
        vals, idx = jax.lax.top_k(logits[:, s : s + n].reshape(B, S, n), k)
        w = jax.nn.softmax(vals, axis=-1)
        outs += [w, idx]
    return tuple(outs)
